# baseline (speedup 1.0000x reference)
import functools

import jax
import jax.numpy as jnp
from jax import lax
from jax.experimental import pallas as pl
from jax.experimental.pallas import tpu as pltpu
from jax.experimental.pallas import tpu_sc as plsc

VOCAB = 1000
B, S = 1024, 50
ROWS = B * S
NC, NS, L = 2, 16, 16
NW = NC * NS
ROWS_PER_W = ROWS // NW
C = 100
NZDMA = ROWS_PER_W // C
ZWORDS = C * VOCAB
IB = 80
NIDMA = ROWS_PER_W // IB
ZERO_UNROLL = 10


def _sc_onehot(pos):
    mesh = plsc.VectorSubcoreMesh(core_axis_name="c", subcore_axis_name="s")

    @functools.partial(
        pl.kernel,
        mesh=mesh,
        out_type=jax.ShapeDtypeStruct((ROWS * VOCAB,), jnp.float32),
        scratch_types=[
            pltpu.VMEM((ROWS_PER_W,), jnp.int32),
            pltpu.VMEM((ZWORDS,), jnp.float32),
            pltpu.VMEM((NIDMA, IB), jnp.int32),
            pltpu.VMEM((IB,), jnp.float32),
            pltpu.SemaphoreType.DMA,
            pltpu.SemaphoreType.DMA,
        ],
        compiler_params=pltpu.CompilerParams(needs_layout_passes=False),
    )
    def k(pos_hbm, out_hbm, pos_v, zer_v, idx_v, ones_v, sem, psem):
        wid = lax.axis_index("s") * NC + lax.axis_index("c")
        base_row = wid * ROWS_PER_W
        ph = pltpu.async_copy(
            pos_hbm.at[pl.ds(base_row, ROWS_PER_W)], pos_v, psem
        )

        zeros16 = jnp.zeros((L,), jnp.float32)
        ones16 = jnp.ones((L,), jnp.float32)

        def zero_body(i, c):
            for u in range(ZERO_UNROLL):
                zer_v[pl.ds((i * ZERO_UNROLL + u) * L, L)] = zeros16
            return c

        lax.fori_loop(0, ZWORDS // (L * ZERO_UNROLL), zero_body, 0)

        zh = []
        for t in range(NZDMA):
            zh.append(
                pltpu.async_copy(
                    zer_v,
                    out_hbm.at[pl.ds((base_row + t * C) * VOCAB, ZWORDS)],
                    sem,
                )
            )

        for c in range(IB // L):
            ones_v[pl.ds(c * L, L)] = ones16

        ph.wait()
        lane = lax.iota(jnp.int32, L)
        for j in range(NIDMA):
            for g in range(IB // L):
                r0 = j * IB + g * L
                p16 = pos_v[pl.ds(r0, L)]
                idx_v[j, pl.ds(g * L, L)] = (base_row + r0 + lane) * VOCAB + p16

        for h in zh:
            h.wait()
        sh = []
        for j in range(NIDMA):
            sh.append(pltpu.async_copy(ones_v, out_hbm.at[idx_v.at[j]], sem))
        for h in sh:
            h.wait()

    return k(pos)


def kernel(x):
    del x
    pos = jax.random.randint(
        jax.random.key(42), (B, S), 1, 6, dtype=jnp.int32
    ).reshape(-1)
    out = _sc_onehot(pos)
    return out.reshape(B, S, VOCAB)


# speedup vs baseline: 3.0297x; 3.0297x over previous
import functools

import jax
import jax.numpy as jnp
from jax import lax
from jax.experimental import pallas as pl
from jax.experimental.pallas import tpu as pltpu
from jax.experimental.pallas import tpu_sc as plsc

VOCAB = 1000
B, S = 1024, 50
ROWS = B * S
WORDS = ROWS * VOCAB
NC, NS, L = 2, 16, 16
NW = NC * NS
WPW = WORDS // NW
ZROWS = 100
ZWORDS = ZROWS * VOCAB
NZDMA = WPW // ZWORDS
TOK_PER_W = ROWS // NW
IB = 80
NIDMA = TOK_PER_W // IB
ZERO_UNROLL = 10
SLICE = VOCAB * B


def _sc_onehot(addr_perm):
    mesh = plsc.VectorSubcoreMesh(core_axis_name="c", subcore_axis_name="s")

    @functools.partial(
        pl.kernel,
        mesh=mesh,
        out_type=jax.ShapeDtypeStruct((WORDS,), jnp.float32),
        scratch_types=[
            pltpu.VMEM((ZWORDS,), jnp.float32),
            pltpu.VMEM((NIDMA, IB), jnp.int32),
            pltpu.VMEM((IB,), jnp.float32),
            pltpu.SemaphoreType.DMA,
            pltpu.SemaphoreType.DMA,
        ],
        compiler_params=pltpu.CompilerParams(needs_layout_passes=False),
    )
    def k(addr_hbm, out_hbm, zer_v, idx_v, ones_v, sem, psem):
        wid = lax.axis_index("c") * NS + lax.axis_index("s")
        base = wid * WPW
        ph = pltpu.async_copy(addr_hbm.at[wid], idx_v, psem)

        zeros16 = jnp.zeros((L,), jnp.float32)
        ones16 = jnp.ones((L,), jnp.float32)

        def zero_body(i, c):
            for u in range(ZERO_UNROLL):
                zer_v[pl.ds((i * ZERO_UNROLL + u) * L, L)] = zeros16
            return c

        lax.fori_loop(0, ZWORDS // (L * ZERO_UNROLL), zero_body, 0)

        zh = []
        for t in range(NZDMA):
            zh.append(
                pltpu.async_copy(
                    zer_v, out_hbm.at[pl.ds(base + t * ZWORDS, ZWORDS)], sem
                )
            )

        for c in range(IB // L):
            ones_v[pl.ds(c * L, L)] = ones16

        ph.wait()
        for h in zh:
            h.wait()
        plsc.subcore_barrier()
        sh = []
        for j in range(NIDMA):
            sh.append(pltpu.async_copy(ones_v, out_hbm.at[idx_v.at[j]], sem))
        for h in sh:
            h.wait()

    return k(addr_perm)


def kernel(x):
    del x
    pos = jax.random.randint(
        jax.random.key(42), (B, S), 1, 6, dtype=jnp.int32
    )
    b = jnp.arange(B, dtype=jnp.int32)[:, None]
    s = jnp.arange(S, dtype=jnp.int32)[None, :]
    addr = s * SLICE + (b // 128) * 1024 + pos * 128 + (b % 128)
    addr_perm = (
        addr.reshape(NS, 64, NC, S // NC)
        .transpose(2, 0, 3, 1)
        .reshape(NW, NIDMA, IB)
    )
    out = _sc_onehot(addr_perm)
    g = out.reshape(S, VOCAB // 8, B // 128, 8, 128)
    return g.transpose(2, 4, 0, 1, 3).reshape(B, S, VOCAB)


# speedup vs baseline: 6.9772x; 2.3029x over previous
import functools

import jax
import jax.numpy as jnp
from jax import lax
from jax.experimental import pallas as pl
from jax.experimental.pallas import tpu as pltpu
from jax.experimental.pallas import tpu_sc as plsc

VOCAB = 1000
B, S = 1024, 50
WORDS = B * S * VOCAB
NC, NS, L = 2, 16, 16
NW = NC * NS
WPW = WORDS // NW
SLICE = VOCAB * B
BLK = 8 * B
ZWORDS = 100000
NZDMA = WPW // ZWORDS
ZERO_UNROLL = 10


def _sc_onehot(pos_t):
    mesh = plsc.VectorSubcoreMesh(core_axis_name="c", subcore_axis_name="s")

    @functools.partial(
        pl.kernel,
        mesh=mesh,
        out_type=jax.ShapeDtypeStruct((WORDS,), jnp.float32),
        scratch_types=[
            pltpu.VMEM((ZWORDS,), jnp.float32),
            pltpu.VMEM((BLK,), jnp.float32),
            pltpu.VMEM((BLK,), jnp.float32),
            pltpu.VMEM((B,), jnp.int32),
            pltpu.VMEM((B,), jnp.int32),
            pltpu.SemaphoreType.DMA,
            pltpu.SemaphoreType.DMA,
        ],
        compiler_params=pltpu.CompilerParams(needs_layout_passes=False),
    )
    def k(pos_hbm, out_hbm, zer_v, blka_v, blkb_v, posa_v, posb_v, sem, psem):
        wid = lax.axis_index("c") * NS + lax.axis_index("s")
        base = wid * WPW
        s1 = (base + SLICE - 1) // SLICE
        s2 = jnp.where((s1 + 1) * SLICE < base + WPW, s1 + 1, s1)
        pha = pltpu.async_copy(pos_hbm.at[s1], posa_v, psem)
        phb = pltpu.async_copy(pos_hbm.at[s2], posb_v, psem)

        zeros16 = jnp.zeros((L,), jnp.float32)

        def zero_body(i, c):
            for u in range(ZERO_UNROLL):
                zer_v[pl.ds((i * ZERO_UNROLL + u) * L, L)] = zeros16
            return c

        lax.fori_loop(0, ZWORDS // (L * ZERO_UNROLL), zero_body, 0)

        zh = []
        for t in range(NZDMA):
            zh.append(
                pltpu.async_copy(
                    zer_v, out_hbm.at[pl.ds(base + t * ZWORDS, ZWORDS)], sem
                )
            )

        def bzero_body(i, c):
            for u in range(8):
                off = (i * 8 + u) * L
                blka_v[pl.ds(off, L)] = zeros16
                blkb_v[pl.ds(off, L)] = zeros16
            return c

        lax.fori_loop(0, BLK // (L * 8), bzero_body, 0)

        pha.wait()
        phb.wait()

        def build_body(bt, c):
            for g in range(8):
                b16 = bt * 128 + g * L
                pa16 = posa_v[pl.ds(b16, L)]
                pb16 = posb_v[pl.ds(b16, L)]
                for vi in range(1, 6):
                    off = vi * 128 + g * L
                    blka_v[pl.ds(bt * 1024 + off, L)] = jnp.where(
                        pa16 == vi, 1.0, 0.0
                    ).astype(jnp.float32)
                    blkb_v[pl.ds(bt * 1024 + off, L)] = jnp.where(
                        pb16 == vi, 1.0, 0.0
                    ).astype(jnp.float32)
            return c

        lax.fori_loop(0, 8, build_body, 0)

        for h in zh:
            h.wait()
        bha = pltpu.async_copy(blka_v, out_hbm.at[pl.ds(s1 * SLICE, BLK)], sem)
        bhb = pltpu.async_copy(blkb_v, out_hbm.at[pl.ds(s2 * SLICE, BLK)], sem)
        bha.wait()
        bhb.wait()

    return k(pos_t)


def kernel(x):
    del x
    pos = jax.random.randint(
        jax.random.key(42), (B, S), 1, 6, dtype=jnp.int32
    )
    out = _sc_onehot(pos.T.reshape(S, B))
    g = out.reshape(S, VOCAB // 8, B // 128, 8, 128)
    return g.transpose(2, 4, 0, 1, 3).reshape(B, S, VOCAB)


# speedup vs baseline: 7.1597x; 1.0262x over previous
import functools

import jax
import jax.numpy as jnp
from jax import lax
from jax.experimental import pallas as pl
from jax.experimental.pallas import tpu as pltpu
from jax.experimental.pallas import tpu_sc as plsc

VOCAB = 1000
B, S = 1024, 50
WORDS = B * S * VOCAB
NC, NS, L = 2, 16, 16
NW = NC * NS
WPW = WORDS // NW
SLICE = VOCAB * B
BLK = 8 * B
ZWORDS = 32000
NZDMA = WPW // ZWORDS
ZERO_UNROLL = 10


def _sc_onehot(pos_t):
    mesh = plsc.VectorSubcoreMesh(core_axis_name="c", subcore_axis_name="s")

    @functools.partial(
        pl.kernel,
        mesh=mesh,
        out_type=jax.ShapeDtypeStruct((WORDS,), jnp.float32),
        scratch_types=[
            pltpu.VMEM((ZWORDS,), jnp.float32),
            pltpu.VMEM((BLK,), jnp.float32),
            pltpu.VMEM((BLK,), jnp.float32),
            pltpu.VMEM((B,), jnp.int32),
            pltpu.VMEM((B,), jnp.int32),
            pltpu.SemaphoreType.DMA,
            pltpu.SemaphoreType.DMA,
        ],
        compiler_params=pltpu.CompilerParams(needs_layout_passes=False),
    )
    def k(pos_hbm, out_hbm, zer_v, blka_v, blkb_v, posa_v, posb_v, sem, psem):
        wid = lax.axis_index("c") * NS + lax.axis_index("s")
        base = wid * WPW
        s1 = (base + SLICE - 1) // SLICE
        s2 = jnp.where((s1 + 1) * SLICE < base + WPW, s1 + 1, s1)
        pha = pltpu.async_copy(pos_hbm.at[s1], posa_v, psem)
        phb = pltpu.async_copy(pos_hbm.at[s2], posb_v, psem)

        zeros16 = jnp.zeros((L,), jnp.float32)

        def zero_body(i, c):
            for u in range(ZERO_UNROLL):
                zer_v[pl.ds((i * ZERO_UNROLL + u) * L, L)] = zeros16
            return c

        lax.fori_loop(0, ZWORDS // (L * ZERO_UNROLL), zero_body, 0)

        zh = []
        for t in range(NZDMA):
            zh.append(
                pltpu.async_copy(
                    zer_v, out_hbm.at[pl.ds(base + t * ZWORDS, ZWORDS)], sem
                )
            )

        def bzero_body(i, c):
            for u in range(8):
                off = (i * 8 + u) * L
                blka_v[pl.ds(off, L)] = zeros16
                blkb_v[pl.ds(off, L)] = zeros16
            return c

        lax.fori_loop(0, BLK // (L * 8), bzero_body, 0)

        pha.wait()
        phb.wait()

        def build_body(bt, c):
            for g in range(8):
                b16 = bt * 128 + g * L
                pa16 = posa_v[pl.ds(b16, L)]
                pb16 = posb_v[pl.ds(b16, L)]
                for vi in range(1, 6):
                    off = vi * 128 + g * L
                    blka_v[pl.ds(bt * 1024 + off, L)] = jnp.where(
                        pa16 == vi, 1.0, 0.0
                    ).astype(jnp.float32)
                    blkb_v[pl.ds(bt * 1024 + off, L)] = jnp.where(
                        pb16 == vi, 1.0, 0.0
                    ).astype(jnp.float32)
            return c

        lax.fori_loop(0, 8, build_body, 0)

        for h in zh:
            h.wait()
        bha = pltpu.async_copy(blka_v, out_hbm.at[pl.ds(s1 * SLICE, BLK)], sem)
        bhb = pltpu.async_copy(blkb_v, out_hbm.at[pl.ds(s2 * SLICE, BLK)], sem)
        bha.wait()
        bhb.wait()

    return k(pos_t)


def kernel(x):
    del x
    pos = jax.random.randint(
        jax.random.key(42), (B, S), 1, 6, dtype=jnp.int32
    )
    out = _sc_onehot(pos.T.reshape(S, B))
    g = out.reshape(S, VOCAB // 8, B // 128, 8, 128)
    return g.transpose(2, 4, 0, 1, 3).reshape(B, S, VOCAB)


# speedup vs baseline: 7.1937x; 1.0047x over previous
import functools

import jax
import jax.numpy as jnp
from jax import lax
from jax.experimental import pallas as pl
from jax.experimental.pallas import tpu as pltpu
from jax.experimental.pallas import tpu_sc as plsc

VOCAB = 1000
B, S = 1024, 50
WORDS = B * S * VOCAB
NC, NS, L = 2, 16, 16
NW = NC * NS
WPW = WORDS // NW
SLICE = VOCAB * B
BLK = 8 * B
ZWORDS = 16000
NZDMA = WPW // ZWORDS
ZERO_UNROLL = 10


def _sc_onehot(pos_t):
    mesh = plsc.VectorSubcoreMesh(core_axis_name="c", subcore_axis_name="s")

    @functools.partial(
        pl.kernel,
        mesh=mesh,
        out_type=jax.ShapeDtypeStruct((WORDS,), jnp.float32),
        scratch_types=[
            pltpu.VMEM((ZWORDS,), jnp.float32),
            pltpu.VMEM((BLK,), jnp.float32),
            pltpu.VMEM((BLK,), jnp.float32),
            pltpu.VMEM((B,), jnp.int32),
            pltpu.VMEM((B,), jnp.int32),
            pltpu.SemaphoreType.DMA,
            pltpu.SemaphoreType.DMA,
        ],
        compiler_params=pltpu.CompilerParams(needs_layout_passes=False),
    )
    def k(pos_hbm, out_hbm, zer_v, blka_v, blkb_v, posa_v, posb_v, sem, psem):
        wid = lax.axis_index("c") * NS + lax.axis_index("s")
        base = wid * WPW
        s1 = (base + SLICE - 1) // SLICE
        s2 = jnp.where((s1 + 1) * SLICE < base + WPW, s1 + 1, s1)
        pha = pltpu.async_copy(pos_hbm.at[s1], posa_v, psem)
        phb = pltpu.async_copy(pos_hbm.at[s2], posb_v, psem)

        zeros16 = jnp.zeros((L,), jnp.float32)

        def zero_body(i, c):
            for u in range(ZERO_UNROLL):
                zer_v[pl.ds((i * ZERO_UNROLL + u) * L, L)] = zeros16
            return c

        lax.fori_loop(0, ZWORDS // (L * ZERO_UNROLL), zero_body, 0)

        zh = []
        for t in range(NZDMA):
            zh.append(
                pltpu.async_copy(
                    zer_v, out_hbm.at[pl.ds(base + t * ZWORDS, ZWORDS)], sem
                )
            )

        def bzero_body(i, c):
            for u in range(8):
                off = (i * 8 + u) * L
                blka_v[pl.ds(off, L)] = zeros16
                blkb_v[pl.ds(off, L)] = zeros16
            return c

        lax.fori_loop(0, BLK // (L * 8), bzero_body, 0)

        pha.wait()
        phb.wait()

        def build_body(bt, c):
            for g in range(8):
                b16 = bt * 128 + g * L
                pa16 = posa_v[pl.ds(b16, L)]
                pb16 = posb_v[pl.ds(b16, L)]
                for vi in range(1, 6):
                    off = vi * 128 + g * L
                    blka_v[pl.ds(bt * 1024 + off, L)] = jnp.where(
                        pa16 == vi, 1.0, 0.0
                    ).astype(jnp.float32)
                    blkb_v[pl.ds(bt * 1024 + off, L)] = jnp.where(
                        pb16 == vi, 1.0, 0.0
                    ).astype(jnp.float32)
            return c

        lax.fori_loop(0, 8, build_body, 0)

        for h in zh:
            h.wait()
        bha = pltpu.async_copy(blka_v, out_hbm.at[pl.ds(s1 * SLICE, BLK)], sem)
        bhb = pltpu.async_copy(blkb_v, out_hbm.at[pl.ds(s2 * SLICE, BLK)], sem)
        bha.wait()
        bhb.wait()

    return k(pos_t)


def kernel(x):
    del x
    pos = jax.random.randint(
        jax.random.key(42), (B, S), 1, 6, dtype=jnp.int32
    )
    out = _sc_onehot(pos.T.reshape(S, B))
    g = out.reshape(S, VOCAB // 8, B // 128, 8, 128)
    return g.transpose(2, 4, 0, 1, 3).reshape(B, S, VOCAB)
